# baseline (speedup 1.0000x reference)
import functools

import jax
import jax.numpy as jnp
from jax import lax
from jax.experimental import pallas as pl
from jax.experimental.pallas import tpu as pltpu
from jax.experimental.pallas import tpu_sc as plsc

IM_H = 224
IM_W = 224
NVOX = 48
FH = 56
FW = 56
FDIM = 32
NR = 8
PLANE = NVOX * NVOX
NW = 32
PLANES_PER_W = NVOX // 2
CH_PER_W = FDIM // 2
ROWS_PER_CHUNK = 8
CHUNKS_PER_PLANE = NVOX // ROWS_PER_CHUNK
FSLICE = CH_PER_W * FH * FW


def _splat_body(feats_hbm, params_hbm, out_hbm, fv, pv, st_a, st_b,
                sem_a, sem_b):
    c = lax.axis_index("c")
    s = lax.axis_index("s")
    wid = s * 2 + c
    r = lax.shift_right_logical(wid, 2)
    ihalf = lax.bitwise_and(lax.shift_right_logical(wid, 1), 1)
    chalf = lax.bitwise_and(wid, 1)

    pltpu.sync_copy(params_hbm.at[wid], pv)
    pltpu.sync_copy(feats_hbm.at[r, pl.ds(chalf * FSLICE, FSLICE)], fv)

    bx, by, bz = pv[0], pv[1], pv[2]
    skx, sky, skz = pv[3], pv[4], pv[5]
    sjx, sjy, sjz = pv[6], pv[7], pv[8]
    six, siy, siz = pv[9], pv[10], pv[11]

    rsz = jnp.float32(float(FH) / IM_H)
    xmax = jnp.float32(FW - 1)
    ymax = jnp.float32(FH - 1)
    one = jnp.float32(1.0)
    i0 = ihalf * PLANES_PER_W
    ch0 = chalf * CH_PER_W

    def row_body_for(stage):
        def row_body(jj, rr):
            rx, ry, rz = rr
            xh, yh, zh = rx, ry, rz
            for g in range(3):
                x = (xh / zh) * rsz
                y = (yh / zh) * rsz
                x = jnp.clip(x, 0.0, xmax)
                y = jnp.clip(y, 0.0, ymax)
                x0i = x.astype(jnp.int32)
                y0i = y.astype(jnp.int32)
                x0i = jnp.clip(x0i, 0, FW - 1)
                y0i = jnp.clip(y0i, 0, FH - 1)
                x0f = x0i.astype(jnp.float32)
                y0f = y0i.astype(jnp.float32)
                x1f = jnp.minimum(x0f + one, xmax)
                y1f = jnp.minimum(y0f + one, ymax)
                y1i = y1f.astype(jnp.int32)
                dx1 = x1f - x
                dx0 = x - x0f
                dy1 = y1f - y
                dy0 = y - y0f
                wa = dx1 * dy1
                wb = dx1 * dy0
                wc = dx0 * dy1
                wd = dx0 * dy0
                t0 = y0i * FW
                t1 = y1i * FW
                ja = t0 + x0i
                jb = t1 + x0i
                hi_mask = jnp.full((16,), -65536, jnp.int32)
                off = g * 16
                for ch in range(CH_PER_W):
                    if ch:
                        co = jnp.full((16,), ch * FH * FW, jnp.int32)
                        ka, kb = ja + co, jb + co
                    else:
                        ka, kb = ja, jb
                    wta = plsc.load_gather(fv, [ka])
                    wtb = plsc.load_gather(fv, [kb])
                    ga = plsc.bitcast(lax.shift_left(wta, 16), jnp.float32)
                    gc = plsc.bitcast(jnp.bitwise_and(wta, hi_mask),
                                      jnp.float32)
                    gb = plsc.bitcast(lax.shift_left(wtb, 16), jnp.float32)
                    gd = plsc.bitcast(jnp.bitwise_and(wtb, hi_mask),
                                      jnp.float32)
                    acc = wa * ga + wb * gb + wc * gc + wd * gd
                    stage[ch, jj, pl.ds(off, 16)] = acc
                if g < 2:
                    xh = xh + skx
                    yh = yh + sky
                    zh = zh + skz
            return (rx + sjx, ry + sjy, rz + sjz)
        return row_body

    def plane_body(i, plane):
        px, py, pz = plane

        def pair_body(cp, rowc):
            not_first = (i * (CHUNKS_PER_PLANE // 2) + cp) > 0
            i_abs = i0 + i
            j0_a = (2 * cp) * ROWS_PER_CHUNK
            j0_b = j0_a + ROWS_PER_CHUNK
            cpy_a = pltpu.make_async_copy(
                st_a,
                out_hbm.at[r, pl.ds(ch0, CH_PER_W), i_abs,
                           pl.ds(j0_a, ROWS_PER_CHUNK), :],
                sem_a)
            cpy_b = pltpu.make_async_copy(
                st_b,
                out_hbm.at[r, pl.ds(ch0, CH_PER_W), i_abs,
                           pl.ds(j0_b, ROWS_PER_CHUNK), :],
                sem_b)

            @pl.when(not_first)
            def _():
                cpy_a.wait()

            rowc = lax.fori_loop(0, ROWS_PER_CHUNK, row_body_for(st_a), rowc)
            cpy_a.start()

            @pl.when(not_first)
            def _():
                cpy_b.wait()

            rowc = lax.fori_loop(0, ROWS_PER_CHUNK, row_body_for(st_b), rowc)
            cpy_b.start()
            return rowc

        lax.fori_loop(0, CHUNKS_PER_PLANE // 2, pair_body, (px, py, pz))
        return (px + six, py + siy, pz + siz)

    lax.fori_loop(0, PLANES_PER_W, plane_body, (bx, by, bz))
    pltpu.make_async_copy(
        st_a, out_hbm.at[r, pl.ds(ch0, CH_PER_W), 0,
                         pl.ds(0, ROWS_PER_CHUNK), :], sem_a).wait()
    pltpu.make_async_copy(
        st_b, out_hbm.at[r, pl.ds(ch0, CH_PER_W), 0,
                         pl.ds(0, ROWS_PER_CHUNK), :], sem_b).wait()


@jax.jit
def _proj_splat(feats, Kcam, vmax, vmin):
    P = Kcam.reshape(NR, 3, 4).astype(jnp.float32)
    bidx = jnp.arange(NR, dtype=jnp.int32) // 4
    vmin_b = vmin[bidx].astype(jnp.float32)
    vmax_b = vmax[bidx].astype(jnp.float32)
    stp = (vmax_b - vmin_b) / jnp.float32(NVOX - 1)
    A = (P[:, :, 0] * vmin_b[:, None, 0] + P[:, :, 1] * vmin_b[:, None, 1]
         + P[:, :, 2] * vmin_b[:, None, 2] + P[:, :, 3])
    Bi = P[:, :, 0] * stp[:, None, 0]
    Bj = P[:, :, 1] * stp[:, None, 1]
    Bk = P[:, :, 2] * stp[:, None, 2]
    lane = jnp.arange(16, dtype=jnp.float32)
    iv = jnp.array([0, 0, 1, 1], dtype=jnp.float32) * PLANES_PER_W
    base = (A[:, None, :, None] + Bi[:, None, :, None] * iv[None, :, None, None]
            + Bk[:, None, :, None] * lane[None, None, None, :])
    base = base.reshape(NW, 3, 16)
    tile = lambda B: jnp.broadcast_to(
        B[:, None, :, None], (NR, 4, 3, 16)).reshape(NW, 3, 16)
    params = jnp.concatenate(
        [base, tile(16.0 * Bk), tile(Bj), tile(Bi)], axis=1)
    fb = feats.astype(jnp.bfloat16)
    xr = jnp.minimum(jnp.arange(FW) + 1, FW - 1)
    fr = fb[..., xr]
    lo = lax.bitcast_convert_type(fb, jnp.uint16).astype(jnp.uint32)
    hi = lax.bitcast_convert_type(fr, jnp.uint16).astype(jnp.uint32)
    packed = lax.bitcast_convert_type(lo | (hi << 16), jnp.int32)
    feats_flat = packed.reshape(NR, FDIM * FH * FW)

    mesh = plsc.VectorSubcoreMesh(core_axis_name="c", subcore_axis_name="s")
    run = functools.partial(
        pl.kernel,
        mesh=mesh,
        compiler_params=pltpu.CompilerParams(needs_layout_passes=False),
        out_type=jax.ShapeDtypeStruct((NR, FDIM, NVOX, NVOX, NVOX),
                                      jnp.float32),
        scratch_types=[
            pltpu.VMEM((FSLICE,), jnp.int32),
            pltpu.VMEM((12, 16), jnp.float32),
            pltpu.VMEM((CH_PER_W, ROWS_PER_CHUNK, NVOX), jnp.float32),
            pltpu.VMEM((CH_PER_W, ROWS_PER_CHUNK, NVOX), jnp.float32),
            pltpu.SemaphoreType.DMA,
            pltpu.SemaphoreType.DMA,
        ],
    )(_splat_body)
    return run(feats_flat, params)


def kernel(feats, Kcam, vmax, vmin, nvox):
    del nvox
    return _proj_splat(feats, Kcam, vmax, vmin)


# speedup vs baseline: 2.6374x; 2.6374x over previous
import functools

import jax
import jax.numpy as jnp
from jax import lax
from jax.experimental import pallas as pl
from jax.experimental.pallas import tpu as pltpu
from jax.experimental.pallas import tpu_sc as plsc

IM_H = 224
IM_W = 224
NVOX = 48
FH = 56
FW = 56
FDIM = 32
NR = 8
PLANE = NVOX * NVOX
NW = 32
PLANES_PER_W = NVOX // 2
CH_PER_W = FDIM // 2
ROWS_PER_CHUNK = 8
CHUNKS_PER_PLANE = NVOX // ROWS_PER_CHUNK
FSLICE = CH_PER_W * FH * FW


def _splat_body(feats_hbm, params_hbm, out_hbm, fv, pv, st_a, st_b,
                sem_a, sem_b):
    c = lax.axis_index("c")
    s = lax.axis_index("s")
    wid = s * 2 + c
    r = lax.shift_right_logical(wid, 2)
    ihalf = lax.bitwise_and(lax.shift_right_logical(wid, 1), 1)
    chalf = lax.bitwise_and(wid, 1)

    pltpu.sync_copy(params_hbm.at[wid], pv)
    pltpu.sync_copy(feats_hbm.at[r, pl.ds(chalf * FSLICE, FSLICE)], fv)

    bx, by, bz = pv[0], pv[1], pv[2]
    skx, sky, skz = pv[3], pv[4], pv[5]
    sjx, sjy, sjz = pv[6], pv[7], pv[8]
    six, siy, siz = pv[9], pv[10], pv[11]

    rsz = jnp.float32(float(FH) / IM_H)
    xmax = jnp.float32(FW - 1)
    ymax = jnp.float32(FH - 1)
    one = jnp.float32(1.0)
    i0 = ihalf * PLANES_PER_W
    ch0 = chalf * CH_PER_W

    def row_body_for(stage):
        def row_body(jj, rr):
            rx, ry, rz = rr
            xh, yh, zh = rx, ry, rz
            for g in range(3):
                x = (xh / zh) * rsz
                y = (yh / zh) * rsz
                x = jnp.clip(x, 0.0, xmax)
                y = jnp.clip(y, 0.0, ymax)
                x0i = x.astype(jnp.int32)
                y0i = y.astype(jnp.int32)
                x0i = jnp.clip(x0i, 0, FW - 1)
                y0i = jnp.clip(y0i, 0, FH - 1)
                x0f = x0i.astype(jnp.float32)
                y0f = y0i.astype(jnp.float32)
                x1f = jnp.minimum(x0f + one, xmax)
                y1f = jnp.minimum(y0f + one, ymax)
                x1i = x1f.astype(jnp.int32)
                y1i = y1f.astype(jnp.int32)
                dx1 = x1f - x
                dx0 = x - x0f
                dy1 = y1f - y
                dy0 = y - y0f
                wa = dx1 * dy1
                wb = dx1 * dy0
                wc = dx0 * dy1
                wd = dx0 * dy0
                t0 = y0i * FW
                t1 = y1i * FW
                ja = t0 + x0i
                jb = t1 + x0i
                jc = t0 + x1i
                jd = t1 + x1i
                off = g * 16
                accs = []
                for ch in range(CH_PER_W):
                    if ch:
                        co = jnp.full((16,), ch * FH * FW, jnp.int32)
                        ka, kb, kc, kd = ja + co, jb + co, jc + co, jd + co
                    else:
                        ka, kb, kc, kd = ja, jb, jc, jd
                    ga = plsc.load_gather(fv, [ka])
                    gb = plsc.load_gather(fv, [kb])
                    gc = plsc.load_gather(fv, [kc])
                    gd = plsc.load_gather(fv, [kd])
                    accs.append(wa * ga + wb * gb + wc * gc + wd * gd)
                for ch in range(CH_PER_W):
                    stage[ch, jj, pl.ds(off, 16)] = accs[ch]
                if g < 2:
                    xh = xh + skx
                    yh = yh + sky
                    zh = zh + skz
            return (rx + sjx, ry + sjy, rz + sjz)
        return row_body

    def plane_body(i, plane):
        px, py, pz = plane

        def pair_body(cp, rowc):
            not_first = (i * (CHUNKS_PER_PLANE // 2) + cp) > 0
            i_abs = i0 + i
            j0_a = (2 * cp) * ROWS_PER_CHUNK
            j0_b = j0_a + ROWS_PER_CHUNK
            cpy_a = pltpu.make_async_copy(
                st_a,
                out_hbm.at[r, pl.ds(ch0, CH_PER_W), i_abs,
                           pl.ds(j0_a, ROWS_PER_CHUNK), :],
                sem_a)
            cpy_b = pltpu.make_async_copy(
                st_b,
                out_hbm.at[r, pl.ds(ch0, CH_PER_W), i_abs,
                           pl.ds(j0_b, ROWS_PER_CHUNK), :],
                sem_b)

            @pl.when(not_first)
            def _():
                cpy_a.wait()

            rowc = lax.fori_loop(0, ROWS_PER_CHUNK, row_body_for(st_a), rowc)
            cpy_a.start()

            @pl.when(not_first)
            def _():
                cpy_b.wait()

            rowc = lax.fori_loop(0, ROWS_PER_CHUNK, row_body_for(st_b), rowc)
            cpy_b.start()
            return rowc

        lax.fori_loop(0, CHUNKS_PER_PLANE // 2, pair_body, (px, py, pz))
        return (px + six, py + siy, pz + siz)

    lax.fori_loop(0, PLANES_PER_W, plane_body, (bx, by, bz))
    pltpu.make_async_copy(
        st_a, out_hbm.at[r, pl.ds(ch0, CH_PER_W), 0,
                         pl.ds(0, ROWS_PER_CHUNK), :], sem_a).wait()
    pltpu.make_async_copy(
        st_b, out_hbm.at[r, pl.ds(ch0, CH_PER_W), 0,
                         pl.ds(0, ROWS_PER_CHUNK), :], sem_b).wait()


@jax.jit
def _proj_splat(feats, Kcam, vmax, vmin):
    P = Kcam.reshape(NR, 3, 4).astype(jnp.float32)
    bidx = jnp.arange(NR, dtype=jnp.int32) // 4
    vmin_b = vmin[bidx].astype(jnp.float32)
    vmax_b = vmax[bidx].astype(jnp.float32)
    stp = (vmax_b - vmin_b) / jnp.float32(NVOX - 1)
    A = (P[:, :, 0] * vmin_b[:, None, 0] + P[:, :, 1] * vmin_b[:, None, 1]
         + P[:, :, 2] * vmin_b[:, None, 2] + P[:, :, 3])
    Bi = P[:, :, 0] * stp[:, None, 0]
    Bj = P[:, :, 1] * stp[:, None, 1]
    Bk = P[:, :, 2] * stp[:, None, 2]
    lane = jnp.arange(16, dtype=jnp.float32)
    iv = jnp.array([0, 0, 1, 1], dtype=jnp.float32) * PLANES_PER_W
    base = (A[:, None, :, None] + Bi[:, None, :, None] * iv[None, :, None, None]
            + Bk[:, None, :, None] * lane[None, None, None, :])
    base = base.reshape(NW, 3, 16)
    tile = lambda B: jnp.broadcast_to(
        B[:, None, :, None], (NR, 4, 3, 16)).reshape(NW, 3, 16)
    params = jnp.concatenate(
        [base, tile(16.0 * Bk), tile(Bj), tile(Bi)], axis=1)
    feats_flat = feats.astype(jnp.float32).reshape(NR, FDIM * FH * FW)

    mesh = plsc.VectorSubcoreMesh(core_axis_name="c", subcore_axis_name="s")
    run = functools.partial(
        pl.kernel,
        mesh=mesh,
        compiler_params=pltpu.CompilerParams(needs_layout_passes=False),
        out_type=jax.ShapeDtypeStruct((NR, FDIM, NVOX, NVOX, NVOX),
                                      jnp.float32),
        scratch_types=[
            pltpu.VMEM((FSLICE,), jnp.float32),
            pltpu.VMEM((12, 16), jnp.float32),
            pltpu.VMEM((CH_PER_W, ROWS_PER_CHUNK, NVOX), jnp.float32),
            pltpu.VMEM((CH_PER_W, ROWS_PER_CHUNK, NVOX), jnp.float32),
            pltpu.SemaphoreType.DMA,
            pltpu.SemaphoreType.DMA,
        ],
    )(_splat_body)
    return run(feats_flat, params)


def kernel(feats, Kcam, vmax, vmin, nvox):
    del nvox
    return _proj_splat(feats, Kcam, vmax, vmin)


# speedup vs baseline: 2.7583x; 1.0459x over previous
import functools

import jax
import jax.numpy as jnp
from jax import lax
from jax.experimental import pallas as pl
from jax.experimental.pallas import tpu as pltpu
from jax.experimental.pallas import tpu_sc as plsc

IM_H = 224
IM_W = 224
NVOX = 48
FH = 56
FW = 56
FDIM = 32
NR = 8
PLANE = NVOX * NVOX
NW = 32
PLANES_PER_W = NVOX // 2
CH_PER_W = FDIM // 2
ROWS_PER_CHUNK = 8
CHUNKS_PER_PLANE = NVOX // ROWS_PER_CHUNK
FSLICE = CH_PER_W * FH * FW


def _splat_body(feats_hbm, params_hbm, out_hbm, fv, pv, st_a, st_b,
                sem_a, sem_b):
    c = lax.axis_index("c")
    s = lax.axis_index("s")
    wid = s * 2 + c
    r = lax.shift_right_logical(wid, 2)
    ihalf = lax.bitwise_and(lax.shift_right_logical(wid, 1), 1)
    chalf = lax.bitwise_and(wid, 1)

    pltpu.sync_copy(params_hbm.at[wid], pv)
    pltpu.sync_copy(feats_hbm.at[r, pl.ds(chalf * FSLICE, FSLICE)], fv)

    bx, by, bz = pv[0], pv[1], pv[2]
    skx, sky, skz = pv[3], pv[4], pv[5]
    sjx, sjy, sjz = pv[6], pv[7], pv[8]
    six, siy, siz = pv[9], pv[10], pv[11]

    rsz = jnp.float32(float(FH) / IM_H)
    xmax = jnp.float32(FW - 1)
    ymax = jnp.float32(FH - 1)
    one = jnp.float32(1.0)
    i0 = ihalf * PLANES_PER_W
    ch0 = chalf * CH_PER_W

    def row_body_for(stage):
        def row_body(jj, rr):
            rx, ry, rz = rr
            xh, yh, zh = rx, ry, rz
            for g in range(3):
                x = (xh / zh) * rsz
                y = (yh / zh) * rsz
                x = jnp.clip(x, 0.0, xmax)
                y = jnp.clip(y, 0.0, ymax)
                x0i = x.astype(jnp.int32)
                y0i = y.astype(jnp.int32)
                x0i = jnp.clip(x0i, 0, FW - 1)
                y0i = jnp.clip(y0i, 0, FH - 1)
                x0f = x0i.astype(jnp.float32)
                y0f = y0i.astype(jnp.float32)
                x1f = jnp.minimum(x0f + one, xmax)
                y1f = jnp.minimum(y0f + one, ymax)
                x1i = x1f.astype(jnp.int32)
                y1i = y1f.astype(jnp.int32)
                dx1 = x1f - x
                dx0 = x - x0f
                dy1 = y1f - y
                dy0 = y - y0f
                wa = dx1 * dy1
                wb = dx1 * dy0
                wc = dx0 * dy1
                wd = dx0 * dy0
                t0 = y0i * FW
                t1 = y1i * FW
                ja = t0 + x0i
                jb = t1 + x0i
                jc = t0 + x1i
                jd = t1 + x1i
                off = g * 16
                accs = []
                for ch in range(CH_PER_W):
                    if ch:
                        co = jnp.full((16,), ch * FH * FW, jnp.int32)
                        ka, kb, kc, kd = ja + co, jb + co, jc + co, jd + co
                    else:
                        ka, kb, kc, kd = ja, jb, jc, jd
                    ga = plsc.load_gather(fv, [ka])
                    gb = plsc.load_gather(fv, [kb])
                    gc = plsc.load_gather(fv, [kc])
                    gd = plsc.load_gather(fv, [kd])
                    accs.append(wa * ga + wb * gb + wc * gc + wd * gd)
                for ch in range(CH_PER_W):
                    stage[ch, jj, pl.ds(off, 16)] = accs[ch]
                if g < 2:
                    xh = xh + skx
                    yh = yh + sky
                    zh = zh + skz
            return (rx + sjx, ry + sjy, rz + sjz)
        return row_body

    def plane_body(i, plane):
        px, py, pz = plane

        def pair_body(cp, rowc):
            not_first = (i * (CHUNKS_PER_PLANE // 2) + cp) > 0
            i_abs = i0 + i
            j0_a = (2 * cp) * ROWS_PER_CHUNK
            j0_b = j0_a + ROWS_PER_CHUNK
            cpy_a = pltpu.make_async_copy(
                st_a,
                out_hbm.at[r, pl.ds(ch0, CH_PER_W), i_abs,
                           pl.ds(j0_a, ROWS_PER_CHUNK), :],
                sem_a)
            cpy_b = pltpu.make_async_copy(
                st_b,
                out_hbm.at[r, pl.ds(ch0, CH_PER_W), i_abs,
                           pl.ds(j0_b, ROWS_PER_CHUNK), :],
                sem_b)

            @pl.when(not_first)
            def _():
                cpy_a.wait()

            rowc = plsc.parallel_loop(0, ROWS_PER_CHUNK, 1, unroll=2,
                                      carry=rowc)(row_body_for(st_a))
            cpy_a.start()

            @pl.when(not_first)
            def _():
                cpy_b.wait()

            rowc = plsc.parallel_loop(0, ROWS_PER_CHUNK, 1, unroll=2,
                                      carry=rowc)(row_body_for(st_b))
            cpy_b.start()
            return rowc

        lax.fori_loop(0, CHUNKS_PER_PLANE // 2, pair_body, (px, py, pz))
        return (px + six, py + siy, pz + siz)

    lax.fori_loop(0, PLANES_PER_W, plane_body, (bx, by, bz))
    pltpu.make_async_copy(
        st_a, out_hbm.at[r, pl.ds(ch0, CH_PER_W), 0,
                         pl.ds(0, ROWS_PER_CHUNK), :], sem_a).wait()
    pltpu.make_async_copy(
        st_b, out_hbm.at[r, pl.ds(ch0, CH_PER_W), 0,
                         pl.ds(0, ROWS_PER_CHUNK), :], sem_b).wait()


@jax.jit
def _proj_splat(feats, Kcam, vmax, vmin):
    P = Kcam.reshape(NR, 3, 4).astype(jnp.float32)
    bidx = jnp.arange(NR, dtype=jnp.int32) // 4
    vmin_b = vmin[bidx].astype(jnp.float32)
    vmax_b = vmax[bidx].astype(jnp.float32)
    stp = (vmax_b - vmin_b) / jnp.float32(NVOX - 1)
    A = (P[:, :, 0] * vmin_b[:, None, 0] + P[:, :, 1] * vmin_b[:, None, 1]
         + P[:, :, 2] * vmin_b[:, None, 2] + P[:, :, 3])
    Bi = P[:, :, 0] * stp[:, None, 0]
    Bj = P[:, :, 1] * stp[:, None, 1]
    Bk = P[:, :, 2] * stp[:, None, 2]
    lane = jnp.arange(16, dtype=jnp.float32)
    iv = jnp.array([0, 0, 1, 1], dtype=jnp.float32) * PLANES_PER_W
    base = (A[:, None, :, None] + Bi[:, None, :, None] * iv[None, :, None, None]
            + Bk[:, None, :, None] * lane[None, None, None, :])
    base = base.reshape(NW, 3, 16)
    tile = lambda B: jnp.broadcast_to(
        B[:, None, :, None], (NR, 4, 3, 16)).reshape(NW, 3, 16)
    params = jnp.concatenate(
        [base, tile(16.0 * Bk), tile(Bj), tile(Bi)], axis=1)
    feats_flat = feats.astype(jnp.float32).reshape(NR, FDIM * FH * FW)

    mesh = plsc.VectorSubcoreMesh(core_axis_name="c", subcore_axis_name="s")
    run = functools.partial(
        pl.kernel,
        mesh=mesh,
        compiler_params=pltpu.CompilerParams(needs_layout_passes=False),
        out_type=jax.ShapeDtypeStruct((NR, FDIM, NVOX, NVOX, NVOX),
                                      jnp.float32),
        scratch_types=[
            pltpu.VMEM((FSLICE,), jnp.float32),
            pltpu.VMEM((12, 16), jnp.float32),
            pltpu.VMEM((CH_PER_W, ROWS_PER_CHUNK, NVOX), jnp.float32),
            pltpu.VMEM((CH_PER_W, ROWS_PER_CHUNK, NVOX), jnp.float32),
            pltpu.SemaphoreType.DMA,
            pltpu.SemaphoreType.DMA,
        ],
    )(_splat_body)
    return run(feats_flat, params)


def kernel(feats, Kcam, vmax, vmin, nvox):
    del nvox
    return _proj_splat(feats, Kcam, vmax, vmin)
